# baseline (speedup 1.0000x reference)
import functools

import jax
import jax.numpy as jnp
from jax.experimental import pallas as pl

_B, _T, _N, _C = 1, 12, 10000, 128
_F = 32
_OUT = 12


def _fused_kernel(x_ref, w1a_ref, w1b_ref, w1c_ref, b1_ref,
                  cheb_w_ref, cheb_b_ref,
                  w2a_ref, w2b_ref, w2c_ref, b2_ref,
                  gamma_ref, beta_ref, lin_w_ref, lin_b_ref,
                  y_ref, h_ref):
    nb = x_ref.shape[2]
    X = x_ref[0].reshape(_T * nb, _C)

    dot = functools.partial(jnp.dot, preferred_element_type=jnp.float32)

    b1 = b1_ref[...]
    P = dot(X, w1a_ref[...]) + b1[0]
    Q = jax.nn.sigmoid(dot(X, w1b_ref[...]) + b1[1])
    T0 = jax.nn.relu(P * Q + dot(X, w1c_ref[...]) + b1[2])

    Tg = jax.nn.relu(dot(T0, cheb_w_ref[...]) + cheb_b_ref[...][0])

    b2 = b2_ref[...]
    P2 = dot(Tg, w2a_ref[...]) + b2[0]
    Q2 = jax.nn.sigmoid(dot(Tg, w2b_ref[...]) + b2[1])
    T2 = jax.nn.relu(P2 * Q2 + dot(Tg, w2c_ref[...]) + b2[2])

    T2n = T2.reshape(_T, nb, _F)
    mu = jnp.mean(T2n, axis=(0, 2))
    var = jnp.mean(jnp.square(T2n - mu[None, :, None]), axis=(0, 2))
    rstd = jax.lax.rsqrt(var + 1e-5)
    g = gamma_ref[:, 0] * rstd
    b = beta_ref[:, 0]
    h = (T2n - mu[None, :, None]) * g[None, :, None] + b[None, :, None]

    h_ref[0] = h
    y_ref[...] = dot(jax.nn.relu(h[0]), lin_w_ref[...]) + lin_b_ref[...]


def kernel(x, edge_index, edge_weight,
           tc1_w1, tc1_b1, tc1_w2, tc1_b2, tc1_w3, tc1_b3,
           cheb_w, cheb_b,
           tc2_w1, tc2_b1, tc2_w2, tc2_b2, tc2_w3, tc2_b3,
           bn_gamma, bn_beta, lin_w, lin_b):
    del edge_index, edge_weight
    nb = 1000
    grid = (_N // nb,)

    b1 = jnp.stack([tc1_b1, tc1_b2, tc1_b3])
    b2 = jnp.stack([tc2_b1, tc2_b2, tc2_b3])
    cheb_b2d = cheb_b[None, :]
    lin_b2d = lin_b[None, :]
    gamma = bn_gamma[:, None]
    beta = bn_beta[:, None]

    full = lambda shape: pl.BlockSpec(shape, lambda i: (0,) * len(shape))
    in_specs = [
        pl.BlockSpec((1, _T, nb, _C), lambda i: (0, 0, i, 0)),
        full((_C, _F)), full((_C, _F)), full((_C, _F)), full((3, _F)),
        full((_F, _F)), full((1, _F)),
        full((_F, _F)), full((_F, _F)), full((_F, _F)), full((3, _F)),
        pl.BlockSpec((nb, 1), lambda i: (i, 0)),
        pl.BlockSpec((nb, 1), lambda i: (i, 0)),
        full((_F, _OUT)), full((1, _OUT)),
    ]
    out_specs = [
        pl.BlockSpec((nb, _OUT), lambda i: (i, 0)),
        pl.BlockSpec((1, _T, nb, _F), lambda i: (0, 0, i, 0)),
    ]
    out_shape = [
        jax.ShapeDtypeStruct((_N, _OUT), jnp.float32),
        jax.ShapeDtypeStruct((_B, _T, _N, _F), jnp.float32),
    ]

    y, h = pl.pallas_call(
        _fused_kernel,
        grid=grid,
        in_specs=in_specs,
        out_specs=out_specs,
        out_shape=out_shape,
    )(x, tc1_w1, tc1_w2, tc1_w3, b1, cheb_w, cheb_b2d,
      tc2_w1, tc2_w2, tc2_w3, b2, gamma, beta, lin_w, lin_b2d)
    return (y, h)
